# baseline (speedup 1.0000x reference)
import jax
import jax.numpy as jnp
from jax import lax
from jax.experimental import pallas as pl
from jax.experimental.pallas import tpu as pltpu
from jax.experimental.pallas import tpu_sc as plsc

N = 10000
E = 320000
D = 128
DE = 16
H = 8
DH = 16

HD2 = 64
ACCW = 80
NPAD = 10240
NW = 32
EPW = E // NW
C = 80
NCHUNK = EPW // C
RPS = NPAD // 16


def _split(res):
    return jnp.stack([res[:, :HD2], res[:, HD2:]], axis=0)


def _proj_body(x_ref, qw, qb, kw, kb, vw, vb, q_out, k_out, v_out):
    xb = x_ref[...]
    q_out[...] = _split(
        jnp.dot(xb, qw[...], preferred_element_type=jnp.float32) + qb[...])
    k_out[...] = _split(
        jnp.dot(xb, kw[...], preferred_element_type=jnp.float32) + kb[...])
    v_out[...] = _split(
        jnp.dot(xb, vw[...], preferred_element_type=jnp.float32) + vb[...])


def _proj(x, qw, qb, kw, kb, vw, vb):
    blk = 1000
    grid = (N // blk,)
    w_spec = pl.BlockSpec((D, D), lambda i: (0, 0))
    b_spec = pl.BlockSpec((1, D), lambda i: (0, 0))
    x_spec = pl.BlockSpec((blk, D), lambda i: (i, 0))
    o_spec = pl.BlockSpec((2, blk, HD2), lambda i: (0, i, 0))
    return pl.pallas_call(
        _proj_body,
        grid=grid,
        in_specs=[x_spec, w_spec, b_spec, w_spec, b_spec, w_spec, b_spec],
        out_specs=[o_spec, o_spec, o_spec],
        out_shape=[jax.ShapeDtypeStruct((2, N, HD2), jnp.float32)] * 3,
    )(x, qw, qb, kw, kb, vw, vb)


def _ee_body(ea_ref, w, b, out):
    out[...] = _split(
        (jnp.dot(ea_ref[...], w[...], preferred_element_type=jnp.float32)
         + b[...]) * 0.25)


def _ee(edge_attr, w, b):
    blk = 16000
    grid = (E // blk,)
    return pl.pallas_call(
        _ee_body,
        grid=grid,
        in_specs=[
            pl.BlockSpec((blk, DE), lambda i: (i, 0)),
            pl.BlockSpec((DE, D), lambda i: (0, 0)),
            pl.BlockSpec((1, D), lambda i: (0, 0)),
        ],
        out_specs=pl.BlockSpec((2, blk, HD2), lambda i: (0, i, 0)),
        out_shape=jax.ShapeDtypeStruct((2, E, HD2), jnp.float32),
    )(edge_attr, w, b)


def _edge_body(kh, qh, vh, eeh, srch, dsth, zrows, outh,
               src_all, dst_all, kb0, qb0, vb0, eb0, kb1, qb1, vb1, eb1,
               msg0, msg1, acc, sem0, sem1):
    cid = lax.axis_index("c")
    sid = lax.axis_index("s")
    wid = cid * 16 + sid
    base_e = wid * EPW
    r0 = sid * RPS

    iota16 = lax.iota(jnp.int32, 16)

    pltpu.sync_copy(srch.at[wid], src_all)
    pltpu.sync_copy(dsth.at[wid], dst_all)

    bufs = ((kb0, qb0, vb0, eb0, msg0, sem0),
            (kb1, qb1, vb1, eb1, msg1, sem1))

    for p in range(2):

        def _fire(ci, s):
            kb, qb, vb, eb, _, sem = bufs[s]
            s_idx = src_all.at[ci]
            d_idx = dst_all.at[ci]
            pltpu.async_copy(kh.at[p].at[s_idx], kb, sem)
            pltpu.async_copy(qh.at[p].at[d_idx], qb, sem)
            pltpu.async_copy(vh.at[p].at[s_idx], vb, sem)
            pltpu.async_copy(eeh.at[p].at[pl.ds(base_e + ci * C, C)], eb, sem)

        def _drain(s):
            kb, qb, vb, eb, _, sem = bufs[s]
            for dst_buf in (kb, qb, vb, eb):
                pltpu.make_async_copy(
                    eeh.at[0].at[pl.ds(0, C)], dst_buf, sem).wait()

        def _compute_scatter(ci, s):
            kb, qb, vb, eb, msg, _ = bufs[s]

            def _group(g, carry):
                rows = g * 16 + iota16

                def _head(h, carry2):
                    colb = h * DH
                    acc_v = None
                    for dh in range(DH):
                        cols = jnp.broadcast_to(colb + dh, (16,))
                        kv = plsc.load_gather(kb, [rows, cols])
                        qv = plsc.load_gather(qb, [rows, cols])
                        ev = plsc.load_gather(eb, [rows, cols])
                        t = kv * qv * ev
                        acc_v = t if acc_v is None else acc_v + t
                    sv = jnp.exp(jnp.clip(acc_v, -5.0, 5.0))
                    plsc.store_scatter(
                        msg, [rows, jnp.broadcast_to(HD2 + h, (16,))], sv)
                    for dh in range(DH):
                        cols = jnp.broadcast_to(colb + dh, (16,))
                        vv = plsc.load_gather(vb, [rows, cols])
                        plsc.store_scatter(msg, [rows, cols], vv * sv)
                    return carry2

                return lax.fori_loop(0, H // 2, _head, carry)

            lax.fori_loop(0, C // 16, _group, 0)
            pltpu.sync_copy(msg, acc.at[dst_all.at[ci]], add=True)

        pltpu.sync_copy(zrows, acc.at[pl.ds(r0, RPS)])
        plsc.subcore_barrier()

        _fire(0, 0)

        def _pipe(t, carry):
            c0 = 2 * t
            _fire(c0 + 1, 1)
            _drain(0)
            _compute_scatter(c0, 0)
            _fire(c0 + 2, 0)
            _drain(1)
            _compute_scatter(c0 + 1, 1)
            return carry

        lax.fori_loop(0, (NCHUNK - 1) // 2, _pipe, 0)
        _drain(0)
        _compute_scatter(NCHUNK - 1, 0)

        plsc.subcore_barrier()

        pltpu.sync_copy(acc.at[pl.ds(r0, RPS)],
                        outh.at[cid].at[p].at[pl.ds(r0, RPS)])


def _edge(kh, qh, vh, ee, src, dst, zrows):
    mesh = plsc.VectorSubcoreMesh(core_axis_name="c", subcore_axis_name="s")
    ebuf = pltpu.VMEM((C, HD2), jnp.float32)
    return pl.kernel(
        _edge_body,
        out_type=jax.ShapeDtypeStruct((2, 2, NPAD, ACCW), jnp.float32),
        mesh=mesh,
        scratch_types=[
            pltpu.VMEM((NCHUNK, C), jnp.int32),
            pltpu.VMEM((NCHUNK, C), jnp.int32),
            ebuf, ebuf, ebuf, ebuf, ebuf, ebuf, ebuf, ebuf,
            pltpu.VMEM((C, ACCW), jnp.float32),
            pltpu.VMEM((C, ACCW), jnp.float32),
            pltpu.VMEM_SHARED((NPAD, ACCW), jnp.float32),
            pltpu.SemaphoreType.DMA,
            pltpu.SemaphoreType.DMA,
        ],
        compiler_params=pltpu.CompilerParams(
            needs_layout_passes=False, use_tc_tiling_on_sc=False),
    )(kh, qh, vh, ee, src, dst, zrows)


def _final_body(p_ref, out):
    lo = p_ref[0, 0] + p_ref[1, 0]
    hi = p_ref[0, 1] + p_ref[1, 1]
    blk = lo.shape[0]
    wv = jnp.concatenate([lo[:, :HD2], hi[:, :HD2]], axis=1)
    z = jnp.concatenate([lo[:, HD2:HD2 + 4], hi[:, HD2:HD2 + 4]], axis=1)
    zb = jnp.broadcast_to(z.reshape(blk, H, 1), (blk, H, DH)).reshape(blk, D)
    out[...] = wv / (zb + 1e-6)


def _final(parts):
    blk = 1024
    grid = (NPAD // blk,)
    return pl.pallas_call(
        _final_body,
        grid=grid,
        in_specs=[pl.BlockSpec((2, 2, blk, ACCW), lambda i: (0, 0, i, 0))],
        out_specs=pl.BlockSpec((blk, D), lambda i: (i, 0)),
        out_shape=jax.ShapeDtypeStruct((NPAD, D), jnp.float32),
    )(parts)


def kernel(x, edge_index, edge_attr, batch_vec, Q_W, Q_b, K_W, K_b,
           E_W, E_b, V_W, V_b):
    qh, kh, vh = _proj(x, Q_W, Q_b.reshape(1, D), K_W, K_b.reshape(1, D),
                       V_W, V_b.reshape(1, D))
    ee = _ee(edge_attr, E_W, E_b.reshape(1, D))
    src = edge_index[0].reshape(NW, NCHUNK, C)
    dst = edge_index[1].reshape(NW, NCHUNK, C)
    zrows = jnp.zeros((RPS, ACCW), jnp.float32)
    parts = _edge(kh, qh, vh, ee, src, dst, zrows)
    return _final(parts)[:N]


# speedup vs baseline: 3.6938x; 3.6938x over previous
import jax
import jax.numpy as jnp
from jax import lax
from jax.experimental import pallas as pl
from jax.experimental.pallas import tpu as pltpu
from jax.experimental.pallas import tpu_sc as plsc

N = 10000
E = 320000
D = 128
DE = 16
H = 8
DH = 16

HD2 = 64
ACCW = 80
NPAD = 10240
NW = 32
EPW = E // NW
C = 80
NCHUNK = EPW // C
RPS = NPAD // 16


def _split(res):
    return jnp.stack([res[:, :HD2], res[:, HD2:]], axis=0)


def _proj_body(x_ref, qw, qb, kw, kb, vw, vb, q_out, k_out, v_out):
    xb = x_ref[...]
    q_out[...] = _split(
        jnp.dot(xb, qw[...], preferred_element_type=jnp.float32) + qb[...])
    k_out[...] = _split(
        jnp.dot(xb, kw[...], preferred_element_type=jnp.float32) + kb[...])
    v_out[...] = _split(
        jnp.dot(xb, vw[...], preferred_element_type=jnp.float32) + vb[...])


def _proj(x, qw, qb, kw, kb, vw, vb):
    blk = 1000
    grid = (N // blk,)
    w_spec = pl.BlockSpec((D, D), lambda i: (0, 0))
    b_spec = pl.BlockSpec((1, D), lambda i: (0, 0))
    x_spec = pl.BlockSpec((blk, D), lambda i: (i, 0))
    o_spec = pl.BlockSpec((2, blk, HD2), lambda i: (0, i, 0))
    return pl.pallas_call(
        _proj_body,
        grid=grid,
        in_specs=[x_spec, w_spec, b_spec, w_spec, b_spec, w_spec, b_spec],
        out_specs=[o_spec, o_spec, o_spec],
        out_shape=[jax.ShapeDtypeStruct((2, N, HD2), jnp.float32)] * 3,
    )(x, qw, qb, kw, kb, vw, vb)


def _ee_body(ea_ref, w, b, out):
    out[...] = _split(
        (jnp.dot(ea_ref[...], w[...], preferred_element_type=jnp.float32)
         + b[...]) * 0.25)


def _ee(edge_attr, w, b):
    blk = 16000
    grid = (E // blk,)
    return pl.pallas_call(
        _ee_body,
        grid=grid,
        in_specs=[
            pl.BlockSpec((blk, DE), lambda i: (i, 0)),
            pl.BlockSpec((DE, D), lambda i: (0, 0)),
            pl.BlockSpec((1, D), lambda i: (0, 0)),
        ],
        out_specs=pl.BlockSpec((2, blk, HD2), lambda i: (0, i, 0)),
        out_shape=jax.ShapeDtypeStruct((2, E, HD2), jnp.float32),
    )(edge_attr, w, b)


def _edge_body(kh, qh, vh, eeh, srch, dsth, zrows, outh,
               src_all, dst_all, kb0, qb0, vb0, eb0, kb1, qb1, vb1, eb1,
               msg0, msg1, acc, sem0, sem1):
    cid = lax.axis_index("c")
    sid = lax.axis_index("s")
    wid = cid * 16 + sid
    base_e = wid * EPW
    r0 = sid * RPS

    iota16 = lax.iota(jnp.int32, 16)

    pltpu.sync_copy(srch.at[wid], src_all)
    pltpu.sync_copy(dsth.at[wid], dst_all)

    bufs = ((kb0, qb0, vb0, eb0, msg0, sem0),
            (kb1, qb1, vb1, eb1, msg1, sem1))

    for p in range(2):

        def _fire(ci, s):
            kb, qb, vb, eb, _, sem = bufs[s]
            s_idx = src_all.at[ci]
            d_idx = dst_all.at[ci]
            pltpu.async_copy(kh.at[p].at[s_idx], kb, sem)
            pltpu.async_copy(qh.at[p].at[d_idx], qb, sem)
            pltpu.async_copy(vh.at[p].at[s_idx], vb, sem)
            pltpu.async_copy(eeh.at[p].at[pl.ds(base_e + ci * C, C)], eb, sem)

        def _drain(s):
            kb, qb, vb, eb, _, sem = bufs[s]
            for dst_buf in (kb, qb, vb, eb):
                pltpu.make_async_copy(
                    eeh.at[0].at[pl.ds(0, C)], dst_buf, sem).wait()

        def _compute_scatter(ci, s):
            kb, qb, vb, eb, msg, _ = bufs[s]

            @plsc.parallel_loop(0, C, step=1, unroll=4)
            def _edge_compute(e):
                svec = jnp.zeros((16,), jnp.float32)
                for h in range(H // 2):
                    ks = kb[e, pl.ds(h * DH, DH)]
                    qs = qb[e, pl.ds(h * DH, DH)]
                    es = eb[e, pl.ds(h * DH, DH)]
                    sr = jnp.sum(ks * qs * es)
                    sv = jnp.exp(
                        jnp.clip(jnp.broadcast_to(sr, (16,)), -5.0, 5.0))
                    vs = vb[e, pl.ds(h * DH, DH)]
                    msg[e, pl.ds(h * DH, DH)] = vs * sv
                    svec = jnp.where(iota16 == h, sv, svec)
                msg[e, pl.ds(HD2, 16)] = svec

            pltpu.sync_copy(msg, acc.at[dst_all.at[ci]], add=True)

        pltpu.sync_copy(zrows, acc.at[pl.ds(r0, RPS)])
        plsc.subcore_barrier()

        _fire(0, 0)

        def _pipe(t, carry):
            c0 = 2 * t
            _fire(c0 + 1, 1)
            _drain(0)
            _compute_scatter(c0, 0)
            _fire(c0 + 2, 0)
            _drain(1)
            _compute_scatter(c0 + 1, 1)
            return carry

        lax.fori_loop(0, (NCHUNK - 1) // 2, _pipe, 0)
        _drain(0)
        _compute_scatter(NCHUNK - 1, 0)

        plsc.subcore_barrier()

        pltpu.sync_copy(acc.at[pl.ds(r0, RPS)],
                        outh.at[cid].at[p].at[pl.ds(r0, RPS)])


def _edge(kh, qh, vh, ee, src, dst, zrows):
    mesh = plsc.VectorSubcoreMesh(core_axis_name="c", subcore_axis_name="s")
    ebuf = pltpu.VMEM((C, HD2), jnp.float32)
    return pl.kernel(
        _edge_body,
        out_type=jax.ShapeDtypeStruct((2, 2, NPAD, ACCW), jnp.float32),
        mesh=mesh,
        scratch_types=[
            pltpu.VMEM((NCHUNK, C), jnp.int32),
            pltpu.VMEM((NCHUNK, C), jnp.int32),
            ebuf, ebuf, ebuf, ebuf, ebuf, ebuf, ebuf, ebuf,
            pltpu.VMEM((C, ACCW), jnp.float32),
            pltpu.VMEM((C, ACCW), jnp.float32),
            pltpu.VMEM_SHARED((NPAD, ACCW), jnp.float32),
            pltpu.SemaphoreType.DMA,
            pltpu.SemaphoreType.DMA,
        ],
        compiler_params=pltpu.CompilerParams(
            needs_layout_passes=False, use_tc_tiling_on_sc=False),
    )(kh, qh, vh, ee, src, dst, zrows)


def _final_body(p_ref, out):
    lo = p_ref[0, 0] + p_ref[1, 0]
    hi = p_ref[0, 1] + p_ref[1, 1]
    blk = lo.shape[0]
    wv = jnp.concatenate([lo[:, :HD2], hi[:, :HD2]], axis=1)
    z = jnp.concatenate([lo[:, HD2:HD2 + 4], hi[:, HD2:HD2 + 4]], axis=1)
    zb = jnp.broadcast_to(z.reshape(blk, H, 1), (blk, H, DH)).reshape(blk, D)
    out[...] = wv / (zb + 1e-6)


def _final(parts):
    blk = 1024
    grid = (NPAD // blk,)
    return pl.pallas_call(
        _final_body,
        grid=grid,
        in_specs=[pl.BlockSpec((2, 2, blk, ACCW), lambda i: (0, 0, i, 0))],
        out_specs=pl.BlockSpec((blk, D), lambda i: (i, 0)),
        out_shape=jax.ShapeDtypeStruct((NPAD, D), jnp.float32),
    )(parts)


def kernel(x, edge_index, edge_attr, batch_vec, Q_W, Q_b, K_W, K_b,
           E_W, E_b, V_W, V_b):
    qh, kh, vh = _proj(x, Q_W, Q_b.reshape(1, D), K_W, K_b.reshape(1, D),
                       V_W, V_b.reshape(1, D))
    ee = _ee(edge_attr, E_W, E_b.reshape(1, D))
    src = edge_index[0].reshape(NW, NCHUNK, C)
    dst = edge_index[1].reshape(NW, NCHUNK, C)
    zrows = jnp.zeros((RPS, ACCW), jnp.float32)
    parts = _edge(kh, qh, vh, ee, src, dst, zrows)
    return _final(parts)[:N]


# speedup vs baseline: 5.4827x; 1.4843x over previous
import jax
import jax.numpy as jnp
from jax import lax
from jax.experimental import pallas as pl
from jax.experimental.pallas import tpu as pltpu
from jax.experimental.pallas import tpu_sc as plsc

N = 10000
E = 320000
D = 128
DE = 16
H = 8
DH = 16

HD2 = 64
ACCW = 80
NPAD = 10240
NW = 32
EPW = E // NW
C = 80
NCHUNK = EPW // C
RPS = NPAD // 16


def _split(res):
    return jnp.stack([res[:, :HD2], res[:, HD2:]], axis=0)


def _proj_body(x_ref, qw, qb, kw, kb, vw, vb, q_out, k_out, v_out):
    xb = x_ref[...]
    q_out[...] = _split(
        jnp.dot(xb, qw[...], preferred_element_type=jnp.float32) + qb[...])
    k_out[...] = _split(
        jnp.dot(xb, kw[...], preferred_element_type=jnp.float32) + kb[...])
    v_out[...] = _split(
        jnp.dot(xb, vw[...], preferred_element_type=jnp.float32) + vb[...])


def _proj(x, qw, qb, kw, kb, vw, vb):
    blk = 1000
    grid = (N // blk,)
    w_spec = pl.BlockSpec((D, D), lambda i: (0, 0))
    b_spec = pl.BlockSpec((1, D), lambda i: (0, 0))
    x_spec = pl.BlockSpec((blk, D), lambda i: (i, 0))
    o_spec = pl.BlockSpec((2, blk, HD2), lambda i: (0, i, 0))
    return pl.pallas_call(
        _proj_body,
        grid=grid,
        in_specs=[x_spec, w_spec, b_spec, w_spec, b_spec, w_spec, b_spec],
        out_specs=[o_spec, o_spec, o_spec],
        out_shape=[jax.ShapeDtypeStruct((2, N, HD2), jnp.float32)] * 3,
    )(x, qw, qb, kw, kb, vw, vb)


def _ee_body(ea_ref, w, b, out):
    out[...] = (jnp.dot(ea_ref[...], w[...], preferred_element_type=jnp.float32)
                + b[...]) * 0.25


def _ee(edge_attr, w, b):
    blk = 16000
    grid = (E // blk,)
    return pl.pallas_call(
        _ee_body,
        grid=grid,
        in_specs=[
            pl.BlockSpec((blk, DE), lambda i: (i, 0)),
            pl.BlockSpec((DE, D), lambda i: (0, 0)),
            pl.BlockSpec((1, D), lambda i: (0, 0)),
        ],
        out_specs=pl.BlockSpec((blk, D), lambda i: (i, 0)),
        out_shape=jax.ShapeDtypeStruct((E, D), jnp.float32),
    )(edge_attr, w, b)


def _edge_body(kh, qh, vh, eeh, eidx, zrows, outh,
               src_all, dst_all, kb0, qb0, vb0, eb0, kb1, qb1, vb1, eb1,
               msg0, msg1, acc, sem0, sem1):
    cid = lax.axis_index("c")
    sid = lax.axis_index("s")
    wid = cid * 16 + sid
    base_e = wid * EPW
    r0 = sid * RPS

    iota16 = lax.iota(jnp.int32, 16)

    pltpu.sync_copy(eidx.at[0].at[wid], src_all)
    pltpu.sync_copy(eidx.at[1].at[wid], dst_all)

    bufs = ((kb0, qb0, vb0, eb0, msg0, sem0),
            (kb1, qb1, vb1, eb1, msg1, sem1))

    for p in range(2):

        def _fire(ci, s):
            kb, qb, vb, eb, _, sem = bufs[s]
            s_idx = src_all.at[ci]
            d_idx = dst_all.at[ci]
            pltpu.async_copy(kh.at[p].at[s_idx], kb, sem)
            pltpu.async_copy(qh.at[p].at[d_idx], qb, sem)
            pltpu.async_copy(vh.at[p].at[s_idx], vb, sem)
            pltpu.async_copy(
                eeh.at[pl.ds(base_e + ci * C, C), pl.ds(p * HD2, HD2)],
                eb, sem)

        def _drain(s):
            kb, qb, vb, eb, _, sem = bufs[s]
            for dst_buf in (kb, qb, vb, eb):
                pltpu.make_async_copy(
                    eeh.at[pl.ds(0, C), pl.ds(0, HD2)], dst_buf, sem).wait()

        def _compute_scatter(ci, s):
            kb, qb, vb, eb, msg, _ = bufs[s]

            @plsc.parallel_loop(0, C, step=1, unroll=4)
            def _edge_compute(e):
                svec = jnp.zeros((16,), jnp.float32)
                for h in range(H // 2):
                    ks = kb[e, pl.ds(h * DH, DH)]
                    qs = qb[e, pl.ds(h * DH, DH)]
                    es = eb[e, pl.ds(h * DH, DH)]
                    sr = jnp.sum(ks * qs * es)
                    sv = jnp.exp(
                        jnp.clip(jnp.broadcast_to(sr, (16,)), -5.0, 5.0))
                    vs = vb[e, pl.ds(h * DH, DH)]
                    msg[e, pl.ds(h * DH, DH)] = vs * sv
                    svec = jnp.where(iota16 == h, sv, svec)
                msg[e, pl.ds(HD2, 16)] = svec

            pltpu.sync_copy(msg, acc.at[dst_all.at[ci]], add=True)

        pltpu.sync_copy(zrows, acc.at[pl.ds(r0, RPS)])
        plsc.subcore_barrier()

        _fire(0, 0)

        def _pipe(t, carry):
            c0 = 2 * t
            _fire(c0 + 1, 1)
            _drain(0)
            _compute_scatter(c0, 0)
            _fire(c0 + 2, 0)
            _drain(1)
            _compute_scatter(c0 + 1, 1)
            return carry

        lax.fori_loop(0, (NCHUNK - 1) // 2, _pipe, 0)
        _drain(0)
        _compute_scatter(NCHUNK - 1, 0)

        plsc.subcore_barrier()

        pltpu.sync_copy(acc.at[pl.ds(r0, RPS)],
                        outh.at[cid].at[p].at[pl.ds(r0, RPS)])


def _edge(kh, qh, vh, ee, eidx, zrows):
    mesh = plsc.VectorSubcoreMesh(core_axis_name="c", subcore_axis_name="s")
    ebuf = pltpu.VMEM((C, HD2), jnp.float32)
    return pl.kernel(
        _edge_body,
        out_type=jax.ShapeDtypeStruct((2, 2, NPAD, ACCW), jnp.float32),
        mesh=mesh,
        scratch_types=[
            pltpu.VMEM((NCHUNK, C), jnp.int32),
            pltpu.VMEM((NCHUNK, C), jnp.int32),
            ebuf, ebuf, ebuf, ebuf, ebuf, ebuf, ebuf, ebuf,
            pltpu.VMEM((C, ACCW), jnp.float32),
            pltpu.VMEM((C, ACCW), jnp.float32),
            pltpu.VMEM_SHARED((NPAD, ACCW), jnp.float32),
            pltpu.SemaphoreType.DMA,
            pltpu.SemaphoreType.DMA,
        ],
        compiler_params=pltpu.CompilerParams(
            needs_layout_passes=False, use_tc_tiling_on_sc=False),
    )(kh, qh, vh, ee, eidx, zrows)


def _final_body(p_ref, out):
    lo = p_ref[0, 0] + p_ref[1, 0]
    hi = p_ref[0, 1] + p_ref[1, 1]
    blk = lo.shape[0]
    wv = jnp.concatenate([lo[:, :HD2], hi[:, :HD2]], axis=1)
    z = jnp.concatenate([lo[:, HD2:HD2 + 4], hi[:, HD2:HD2 + 4]], axis=1)
    zb = jnp.broadcast_to(z.reshape(blk, H, 1), (blk, H, DH)).reshape(blk, D)
    out[...] = wv / (zb + 1e-6)


def _final(parts):
    blk = 1000
    grid = (N // blk,)
    return pl.pallas_call(
        _final_body,
        grid=grid,
        in_specs=[pl.BlockSpec((2, 2, blk, ACCW), lambda i: (0, 0, i, 0))],
        out_specs=pl.BlockSpec((blk, D), lambda i: (i, 0)),
        out_shape=jax.ShapeDtypeStruct((N, D), jnp.float32),
    )(parts)


def kernel(x, edge_index, edge_attr, batch_vec, Q_W, Q_b, K_W, K_b,
           E_W, E_b, V_W, V_b):
    qh, kh, vh = _proj(x, Q_W, Q_b.reshape(1, D), K_W, K_b.reshape(1, D),
                       V_W, V_b.reshape(1, D))
    ee = _ee(edge_attr, E_W, E_b.reshape(1, D))
    eidx = edge_index.reshape(2, NW, NCHUNK, C)
    zrows = jnp.zeros((RPS, ACCW), jnp.float32)
    parts = _edge(kh, qh, vh, ee, eidx, zrows)
    return _final(parts)


# speedup vs baseline: 5.6636x; 1.0330x over previous
import jax
import jax.numpy as jnp
from jax import lax
from jax.experimental import pallas as pl
from jax.experimental.pallas import tpu as pltpu
from jax.experimental.pallas import tpu_sc as plsc

N = 10000
E = 320000
D = 128
DE = 16
H = 8
DH = 16

HD2 = 64
ACCW = 80
NPAD = 10240
NW = 32
EPW = E // NW
C = 80
NCHUNK = EPW // C
RPS = NPAD // 16


def _split(res):
    return jnp.stack([res[:, :HD2], res[:, HD2:]], axis=0)


def _proj_body(x_ref, qw, qb, kw, kb, vw, vb, q_out, k_out, v_out):
    xb = x_ref[...]
    q_out[...] = _split(
        jnp.dot(xb, qw[...], preferred_element_type=jnp.float32) + qb[...])
    k_out[...] = _split(
        jnp.dot(xb, kw[...], preferred_element_type=jnp.float32) + kb[...])
    v_out[...] = _split(
        jnp.dot(xb, vw[...], preferred_element_type=jnp.float32) + vb[...])


def _proj(x, qw, qb, kw, kb, vw, vb):
    blk = 1000
    grid = (N // blk,)
    w_spec = pl.BlockSpec((D, D), lambda i: (0, 0))
    b_spec = pl.BlockSpec((1, D), lambda i: (0, 0))
    x_spec = pl.BlockSpec((blk, D), lambda i: (i, 0))
    o_spec = pl.BlockSpec((2, blk, HD2), lambda i: (0, i, 0))
    return pl.pallas_call(
        _proj_body,
        grid=grid,
        in_specs=[x_spec, w_spec, b_spec, w_spec, b_spec, w_spec, b_spec],
        out_specs=[o_spec, o_spec, o_spec],
        out_shape=[jax.ShapeDtypeStruct((2, N, HD2), jnp.float32)] * 3,
    )(x, qw, qb, kw, kb, vw, vb)


def _ee_body(ea_ref, w, b, out):
    out[...] = (jnp.dot(ea_ref[...], w[...], preferred_element_type=jnp.float32)
                + b[...]) * 0.25


def _ee(edge_attr, w, b):
    blk = 16000
    grid = (E // blk,)
    return pl.pallas_call(
        _ee_body,
        grid=grid,
        in_specs=[
            pl.BlockSpec((blk, DE), lambda i: (i, 0)),
            pl.BlockSpec((DE, D), lambda i: (0, 0)),
            pl.BlockSpec((1, D), lambda i: (0, 0)),
        ],
        out_specs=pl.BlockSpec((blk, D), lambda i: (i, 0)),
        out_shape=jax.ShapeDtypeStruct((E, D), jnp.float32),
    )(edge_attr, w, b)


def _edge_body(kh, qh, vh, eeh, eidx, zrows, outh,
               src_all, dst_all, kb0, qb0, vb0, eb0, kb1, qb1, vb1, eb1,
               msg0, msg1, acc, sem0, sem1):
    cid = lax.axis_index("c")
    sid = lax.axis_index("s")
    wid = cid * 16 + sid
    base_e = wid * EPW
    r0 = sid * RPS

    iota16 = lax.iota(jnp.int32, 16)

    pltpu.sync_copy(eidx.at[0].at[pl.ds(base_e, EPW)], src_all)
    pltpu.sync_copy(eidx.at[1].at[pl.ds(base_e, EPW)], dst_all)

    bufs = ((kb0, qb0, vb0, eb0, msg0, sem0),
            (kb1, qb1, vb1, eb1, msg1, sem1))

    for p in range(2):

        def _fire(ci, s):
            kb, qb, vb, eb, _, sem = bufs[s]
            s_idx = src_all.at[pl.ds(ci * C, C)]
            d_idx = dst_all.at[pl.ds(ci * C, C)]
            pltpu.async_copy(kh.at[p].at[s_idx], kb, sem)
            pltpu.async_copy(qh.at[p].at[d_idx], qb, sem)
            pltpu.async_copy(vh.at[p].at[s_idx], vb, sem)
            pltpu.async_copy(
                eeh.at[pl.ds(base_e + ci * C, C), pl.ds(p * HD2, HD2)],
                eb, sem)

        def _drain(s):
            kb, qb, vb, eb, _, sem = bufs[s]
            for dst_buf in (kb, qb, vb, eb):
                pltpu.make_async_copy(
                    eeh.at[pl.ds(0, C), pl.ds(0, HD2)], dst_buf, sem).wait()

        def _compute_scatter(ci, s):
            kb, qb, vb, eb, msg, _ = bufs[s]

            @plsc.parallel_loop(0, C, step=1, unroll=4)
            def _edge_compute(e):
                svec = jnp.zeros((16,), jnp.float32)
                for h in range(H // 2):
                    ks = kb[e, pl.ds(h * DH, DH)]
                    qs = qb[e, pl.ds(h * DH, DH)]
                    es = eb[e, pl.ds(h * DH, DH)]
                    sr = jnp.sum(ks * qs * es)
                    sv = jnp.exp(
                        jnp.clip(jnp.broadcast_to(sr, (16,)), -5.0, 5.0))
                    vs = vb[e, pl.ds(h * DH, DH)]
                    msg[e, pl.ds(h * DH, DH)] = vs * sv
                    svec = jnp.where(iota16 == h, sv, svec)
                msg[e, pl.ds(HD2, 16)] = svec

            pltpu.sync_copy(msg, acc.at[dst_all.at[pl.ds(ci * C, C)]],
                            add=True)

        pltpu.sync_copy(zrows, acc.at[pl.ds(r0, RPS)])
        plsc.subcore_barrier()

        _fire(0, 0)

        def _pipe(t, carry):
            c0 = 2 * t
            _fire(c0 + 1, 1)
            _drain(0)
            _compute_scatter(c0, 0)
            _fire(c0 + 2, 0)
            _drain(1)
            _compute_scatter(c0 + 1, 1)
            return carry

        lax.fori_loop(0, (NCHUNK - 1) // 2, _pipe, 0)
        _drain(0)
        _compute_scatter(NCHUNK - 1, 0)

        plsc.subcore_barrier()

        pltpu.sync_copy(acc.at[pl.ds(r0, RPS)],
                        outh.at[cid].at[p].at[pl.ds(r0, RPS), pl.ds(0, ACCW)])


def _edge(kh, qh, vh, ee, eidx, zrows):
    mesh = plsc.VectorSubcoreMesh(core_axis_name="c", subcore_axis_name="s")
    ebuf = pltpu.VMEM((C, HD2), jnp.float32)
    return pl.kernel(
        _edge_body,
        out_type=jax.ShapeDtypeStruct((2, 2, NPAD, D), jnp.float32),
        mesh=mesh,
        scratch_types=[
            pltpu.VMEM((EPW,), jnp.int32),
            pltpu.VMEM((EPW,), jnp.int32),
            ebuf, ebuf, ebuf, ebuf, ebuf, ebuf, ebuf, ebuf,
            pltpu.VMEM((C, ACCW), jnp.float32),
            pltpu.VMEM((C, ACCW), jnp.float32),
            pltpu.VMEM_SHARED((NPAD, ACCW), jnp.float32),
            pltpu.SemaphoreType.DMA,
            pltpu.SemaphoreType.DMA,
        ],
        compiler_params=pltpu.CompilerParams(
            needs_layout_passes=False, use_tc_tiling_on_sc=False),
    )(kh, qh, vh, ee, eidx, zrows)


def _final_body(p_ref, out):
    lo = p_ref[0, 0, :, 0:ACCW] + p_ref[1, 0, :, 0:ACCW]
    hi = p_ref[0, 1, :, 0:ACCW] + p_ref[1, 1, :, 0:ACCW]
    blk = lo.shape[0]
    wv = jnp.concatenate([lo[:, :HD2], hi[:, :HD2]], axis=1)
    z = jnp.concatenate([lo[:, HD2:HD2 + 4], hi[:, HD2:HD2 + 4]], axis=1)
    zb = jnp.broadcast_to(z.reshape(blk, H, 1), (blk, H, DH)).reshape(blk, D)
    out[...] = wv / (zb + 1e-6)


def _final(parts):
    blk = 1000
    grid = (N // blk,)
    return pl.pallas_call(
        _final_body,
        grid=grid,
        in_specs=[pl.BlockSpec((2, 2, blk, D), lambda i: (0, 0, i, 0))],
        out_specs=pl.BlockSpec((blk, D), lambda i: (i, 0)),
        out_shape=jax.ShapeDtypeStruct((N, D), jnp.float32),
    )(parts)


def kernel(x, edge_index, edge_attr, batch_vec, Q_W, Q_b, K_W, K_b,
           E_W, E_b, V_W, V_b):
    qh, kh, vh = _proj(x, Q_W, Q_b.reshape(1, D), K_W, K_b.reshape(1, D),
                       V_W, V_b.reshape(1, D))
    ee = _ee(edge_attr, E_W, E_b.reshape(1, D))
    zrows = jnp.zeros((RPS, ACCW), jnp.float32)
    parts = _edge(kh, qh, vh, ee, edge_index, zrows)
    return _final(parts)


# speedup vs baseline: 5.7603x; 1.0171x over previous
import jax
import jax.numpy as jnp
from jax import lax
from jax.experimental import pallas as pl
from jax.experimental.pallas import tpu as pltpu
from jax.experimental.pallas import tpu_sc as plsc

N = 10000
E = 320000
D = 128
DE = 16
H = 8
DH = 16

HD2 = 64
ACCW = 80
NPAD = 10240
NW = 32
EPW = E // NW
C = 80
NCHUNK = EPW // C
RPS = NPAD // 16


def _split(res):
    return jnp.stack([res[:, :HD2], res[:, HD2:]], axis=0)


def _proj_body(x_ref, qw, qb, kw, kb, vw, vb, q_out, kv_out):
    xb = x_ref[...]
    q_out[...] = _split(
        jnp.dot(xb, qw[...], preferred_element_type=jnp.float32) + qb[...])
    k = jnp.dot(xb, kw[...], preferred_element_type=jnp.float32) + kb[...]
    v = jnp.dot(xb, vw[...], preferred_element_type=jnp.float32) + vb[...]
    kv_out[...] = jnp.stack(
        [jnp.concatenate([k[:, :HD2], v[:, :HD2]], axis=1),
         jnp.concatenate([k[:, HD2:], v[:, HD2:]], axis=1)], axis=0)


def _proj(x, qw, qb, kw, kb, vw, vb):
    blk = 1000
    grid = (N // blk,)
    w_spec = pl.BlockSpec((D, D), lambda i: (0, 0))
    b_spec = pl.BlockSpec((1, D), lambda i: (0, 0))
    x_spec = pl.BlockSpec((blk, D), lambda i: (i, 0))
    o_spec = pl.BlockSpec((2, blk, HD2), lambda i: (0, i, 0))
    kv_spec = pl.BlockSpec((2, blk, D), lambda i: (0, i, 0))
    return pl.pallas_call(
        _proj_body,
        grid=grid,
        in_specs=[x_spec, w_spec, b_spec, w_spec, b_spec, w_spec, b_spec],
        out_specs=[o_spec, kv_spec],
        out_shape=[jax.ShapeDtypeStruct((2, N, HD2), jnp.float32),
                   jax.ShapeDtypeStruct((2, N, D), jnp.float32)],
    )(x, qw, qb, kw, kb, vw, vb)


def _ee_body(ea_ref, w, b, out):
    out[...] = (jnp.dot(ea_ref[...], w[...], preferred_element_type=jnp.float32)
                + b[...]) * 0.25


def _ee(edge_attr, w, b):
    blk = 16000
    grid = (E // blk,)
    return pl.pallas_call(
        _ee_body,
        grid=grid,
        in_specs=[
            pl.BlockSpec((blk, DE), lambda i: (i, 0)),
            pl.BlockSpec((DE, D), lambda i: (0, 0)),
            pl.BlockSpec((1, D), lambda i: (0, 0)),
        ],
        out_specs=pl.BlockSpec((blk, D), lambda i: (i, 0)),
        out_shape=jax.ShapeDtypeStruct((E, D), jnp.float32),
    )(edge_attr, w, b)


def _edge_body(kvh, qh, eeh, eidx, zrows, outh,
               src_all, dst_all, kvb0, qb0, eb0, kvb1, qb1, eb1,
               msg0, msg1, acc, sem0, sem1):
    cid = lax.axis_index("c")
    sid = lax.axis_index("s")
    wid = cid * 16 + sid
    base_e = wid * EPW
    r0 = sid * RPS

    iota16 = lax.iota(jnp.int32, 16)

    pltpu.sync_copy(eidx.at[0].at[pl.ds(base_e, EPW)], src_all)
    pltpu.sync_copy(eidx.at[1].at[pl.ds(base_e, EPW)], dst_all)

    bufs = ((kvb0, qb0, eb0, msg0, sem0),
            (kvb1, qb1, eb1, msg1, sem1))

    for p in range(2):

        def _fire(ci, s):
            kvb, qb, eb, _, sem = bufs[s]
            s_idx = src_all.at[pl.ds(ci * C, C)]
            d_idx = dst_all.at[pl.ds(ci * C, C)]
            pltpu.async_copy(kvh.at[p].at[s_idx], kvb, sem)
            pltpu.async_copy(qh.at[p].at[d_idx], qb, sem)
            pltpu.async_copy(
                eeh.at[pl.ds(base_e + ci * C, C), pl.ds(p * HD2, HD2)],
                eb, sem)

        def _drain(s):
            kvb, qb, eb, _, sem = bufs[s]
            for dst_buf in (kvb, qb, eb):
                pltpu.make_async_copy(
                    eeh.at[pl.ds(0, C), pl.ds(0, HD2)], dst_buf, sem).wait()

        def _compute_scatter(ci, s):
            kvb, qb, eb, msg, _ = bufs[s]

            @plsc.parallel_loop(0, C, step=1, unroll=4)
            def _edge_compute(e):
                svec = jnp.zeros((16,), jnp.float32)
                for h in range(H // 2):
                    ks = kvb[e, pl.ds(h * DH, DH)]
                    qs = qb[e, pl.ds(h * DH, DH)]
                    es = eb[e, pl.ds(h * DH, DH)]
                    sr = jnp.sum(ks * qs * es)
                    sv = jnp.exp(
                        jnp.clip(jnp.broadcast_to(sr, (16,)), -5.0, 5.0))
                    vs = kvb[e, pl.ds(HD2 + h * DH, DH)]
                    msg[e, pl.ds(h * DH, DH)] = vs * sv
                    svec = jnp.where(iota16 == h, sv, svec)
                msg[e, pl.ds(HD2, 16)] = svec

            pltpu.sync_copy(msg, acc.at[dst_all.at[pl.ds(ci * C, C)]],
                            add=True)

        pltpu.sync_copy(zrows, acc.at[pl.ds(r0, RPS)])
        plsc.subcore_barrier()

        _fire(0, 0)

        def _pipe(t, carry):
            c0 = 2 * t
            _fire(c0 + 1, 1)
            _drain(0)
            _compute_scatter(c0, 0)
            _fire(c0 + 2, 0)
            _drain(1)
            _compute_scatter(c0 + 1, 1)
            return carry

        lax.fori_loop(0, (NCHUNK - 1) // 2, _pipe, 0)
        _drain(0)
        _compute_scatter(NCHUNK - 1, 0)

        plsc.subcore_barrier()

        pltpu.sync_copy(acc.at[pl.ds(r0, RPS)],
                        outh.at[cid].at[p].at[pl.ds(r0, RPS), pl.ds(0, ACCW)])


def _edge(kvh, qh, ee, eidx, zrows):
    mesh = plsc.VectorSubcoreMesh(core_axis_name="c", subcore_axis_name="s")
    hbuf = pltpu.VMEM((C, HD2), jnp.float32)
    kvbuf = pltpu.VMEM((C, D), jnp.float32)
    return pl.kernel(
        _edge_body,
        out_type=jax.ShapeDtypeStruct((2, 2, NPAD, D), jnp.float32),
        mesh=mesh,
        scratch_types=[
            pltpu.VMEM((EPW,), jnp.int32),
            pltpu.VMEM((EPW,), jnp.int32),
            kvbuf, hbuf, hbuf, kvbuf, hbuf, hbuf,
            pltpu.VMEM((C, ACCW), jnp.float32),
            pltpu.VMEM((C, ACCW), jnp.float32),
            pltpu.VMEM_SHARED((NPAD, ACCW), jnp.float32),
            pltpu.SemaphoreType.DMA,
            pltpu.SemaphoreType.DMA,
        ],
        compiler_params=pltpu.CompilerParams(
            needs_layout_passes=False, use_tc_tiling_on_sc=False),
    )(kvh, qh, ee, eidx, zrows)


def _final_body(p_ref, out):
    lo = p_ref[0, 0] + p_ref[1, 0]
    hi = p_ref[0, 1] + p_ref[1, 1]
    blk = lo.shape[0]
    wv = jnp.concatenate([lo[:, :HD2], hi[:, :HD2]], axis=1)
    z = jnp.concatenate([lo[:, HD2:HD2 + 4], hi[:, HD2:HD2 + 4]], axis=1)
    zb = jnp.broadcast_to(z.reshape(blk, H, 1), (blk, H, DH)).reshape(blk, D)
    out[...] = wv / (zb + 1e-6)


def _final(parts):
    blk = 1000
    grid = (N // blk,)
    return pl.pallas_call(
        _final_body,
        grid=grid,
        in_specs=[pl.BlockSpec((2, 2, blk, D), lambda i: (0, 0, i, 0))],
        out_specs=pl.BlockSpec((blk, D), lambda i: (i, 0)),
        out_shape=jax.ShapeDtypeStruct((N, D), jnp.float32),
    )(parts)


def kernel(x, edge_index, edge_attr, batch_vec, Q_W, Q_b, K_W, K_b,
           E_W, E_b, V_W, V_b):
    qh, kvh = _proj(x, Q_W, Q_b.reshape(1, D), K_W, K_b.reshape(1, D),
                    V_W, V_b.reshape(1, D))
    ee = _ee(edge_attr, E_W, E_b.reshape(1, D))
    zrows = jnp.zeros((RPS, ACCW), jnp.float32)
    parts = _edge(kvh, qh, ee, edge_index, zrows)
    return _final(parts)
